# Initial kernel scaffold; baseline (speedup 1.0000x reference)
import jax
import jax.numpy as jnp
from jax.experimental import pallas as pl


def kernel(x, indices, pe):
    raise NotImplementedError("write your pallas kernel here")


# speedup vs baseline: 1.9920x; 1.9920x over previous
import functools

import jax
import jax.numpy as jnp
from jax import lax
from jax.experimental import pallas as pl
from jax.experimental.pallas import tpu as pltpu
from jax.experimental.pallas import tpu_sc as plsc

D = 64
G = 128
NC, NS = 2, 16
NW = NC * NS
LANES = 16


@functools.partial(jax.jit, static_argnames=("n_rows",))
def _sc_add_gather(xf, idxf, pe, *, n_rows):
    groups_per_w = n_rows // (G * NW)

    mesh = plsc.VectorSubcoreMesh(
        core_axis_name="c", subcore_axis_name="s",
        num_cores=NC, num_subcores=NS,
    )

    @functools.partial(
        pl.kernel,
        out_type=jax.ShapeDtypeStruct((n_rows * D,), jnp.float32),
        mesh=mesh,
        scratch_types=[
            pltpu.VMEM((G,), jnp.int32),
            pltpu.VMEM((G, D), jnp.float32),
            pltpu.VMEM((G * D,), jnp.float32),
            pltpu.SemaphoreType.DMA,
        ],
        compiler_params=pltpu.CompilerParams(use_tc_tiling_on_sc=False),
    )
    def body(x_hbm, idx_hbm, pe_hbm, out_hbm, idx_v, rows_v, x_v, sem):
        wid = lax.axis_index("s") * NC + lax.axis_index("c")
        g0 = wid * groups_per_w

        @pl.loop(0, groups_per_w)
        def _group(g):
            grp = g0 + g
            pltpu.sync_copy(idx_hbm.at[pl.ds(grp * G, G)], idx_v)
            gat = pltpu.async_copy(pe_hbm.at[idx_v], rows_v, sem)
            pltpu.sync_copy(x_hbm.at[pl.ds(grp * (G * D), G * D)], x_v)
            gat.wait()

            @pl.loop(0, G)
            def _row(r):
                for j in range(D // LANES):
                    plsc.addupdate(
                        x_v.at[pl.ds(r * D + j * LANES, LANES)],
                        rows_v[r, pl.ds(j * LANES, LANES)],
                    )

            pltpu.sync_copy(x_v, out_hbm.at[pl.ds(grp * (G * D), G * D)])

    return body(xf, idxf, pe)


def kernel(x, indices, pe):
    n_rows = x.shape[0] * x.shape[1]
    xf = x.reshape(n_rows * D)
    idxf = indices.reshape(n_rows)
    out = _sc_add_gather(xf, idxf, pe, n_rows=n_rows)
    return out.reshape(x.shape)


# speedup vs baseline: 2.5044x; 1.2572x over previous
import functools

import jax
import jax.numpy as jnp
from jax import lax
from jax.experimental import pallas as pl
from jax.experimental.pallas import tpu as pltpu
from jax.experimental.pallas import tpu_sc as plsc

D = 64
G = 128
NBUF = 4
NC, NS = 2, 16
NW = NC * NS
LANES = 16


@functools.partial(jax.jit, static_argnames=("n_rows",))
def _sc_add_gather(xf, idxf, pe, *, n_rows):
    rows_pw = n_rows // NW
    ng = rows_pw // G
    assert ng % NBUF == 0

    mesh = plsc.VectorSubcoreMesh(
        core_axis_name="c", subcore_axis_name="s",
        num_cores=NC, num_subcores=NS,
    )

    @functools.partial(
        pl.kernel,
        out_type=jax.ShapeDtypeStruct((n_rows * D,), jnp.float32),
        mesh=mesh,
        scratch_types=[
            pltpu.VMEM((rows_pw,), jnp.int32),
            pltpu.VMEM((NBUF, G * D), jnp.float32),
            pltpu.VMEM((NBUF, G, D), jnp.float32),
            pltpu.SemaphoreType.DMA((NBUF,)),
            pltpu.SemaphoreType.DMA((NBUF,)),
            pltpu.SemaphoreType.DMA((NBUF,)),
        ],
        compiler_params=pltpu.CompilerParams(use_tc_tiling_on_sc=False),
    )
    def body(x_hbm, idx_hbm, pe_hbm, out_hbm, idx_v, x_v, rows_v,
             xsem, gsem, osem):
        wid = lax.axis_index("s") * NC + lax.axis_index("c")
        row0 = wid * rows_pw
        pltpu.sync_copy(idx_hbm.at[pl.ds(row0, rows_pw)], idx_v)

        def in_copies(g, b):
            base = (row0 + g * G) * D
            return (
                pltpu.make_async_copy(
                    x_hbm.at[pl.ds(base, G * D)], x_v.at[b], xsem.at[b]),
                pltpu.make_async_copy(
                    pe_hbm.at[idx_v.at[pl.ds(g * G, G)]], rows_v.at[b],
                    gsem.at[b]),
            )

        def out_copy(g, b):
            base = (row0 + g * G) * D
            return pltpu.make_async_copy(
                x_v.at[b], out_hbm.at[pl.ds(base, G * D)], osem.at[b])

        def issue_in(g, b):
            for c in in_copies(g, b):
                c.start()

        issue_in(0, 0)
        issue_in(1, 1)

        @pl.loop(0, ng, step=NBUF)
        def _ring(g0):
            for b in range(NBUF):
                g = g0 + b
                for c in in_copies(g, b):
                    c.wait()

                @pl.loop(0, G)
                def _row(r):
                    for j in range(D // LANES):
                        plsc.addupdate(
                            x_v.at[b, pl.ds(r * D + j * LANES, LANES)],
                            rows_v[b, r, pl.ds(j * LANES, LANES)],
                        )

                out_copy(g, b).start()

                b2 = (b + 2) % NBUF
                g2 = g + 2

                @pl.when(g2 < ng)
                def _():
                    @pl.when(g2 >= NBUF)
                    def _():
                        out_copy(g2 - NBUF, b2).wait()

                    issue_in(g2, b2)

        out_copy(ng - 2, (ng - 2) % NBUF).wait()
        out_copy(ng - 1, (ng - 1) % NBUF).wait()

    return body(xf, idxf, pe)


def kernel(x, indices, pe):
    n_rows = x.shape[0] * x.shape[1]
    xf = x.reshape(n_rows * D)
    idxf = indices.reshape(n_rows)
    out = _sc_add_gather(xf, idxf, pe, n_rows=n_rows)
    return out.reshape(x.shape)
